# Initial kernel scaffold; baseline (speedup 1.0000x reference)
import jax
import jax.numpy as jnp
from jax.experimental import pallas as pl


def kernel(x, t, loc_tables, time_tables, x_weights, t_weights):
    raise NotImplementedError("write your pallas kernel here")


# speedup vs baseline: 8.4684x; 8.4684x over previous
import functools

import jax
import jax.numpy as jnp
from jax import lax
from jax.experimental import pallas as pl
from jax.experimental.pallas import tpu as pltpu
from jax.experimental.pallas import tpu_sc as plsc

_B, _S = 1024, 200
_L = 4
_VL, _VT = 100000, 512
_D = 32
_N = _B * _S
_NW = 32
_C = 128
_TW = _N // _NW
_NCHW = _TW // _C


def _make_kernel():
    mesh = plsc.VectorSubcoreMesh(core_axis_name="c", subcore_axis_name="s")

    @functools.partial(
        pl.kernel,
        mesh=mesh,
        out_type=jax.ShapeDtypeStruct((_N, 2 * _D), jnp.float32),
        compiler_params=pltpu.CompilerParams(use_tc_tiling_on_sc=False),
        scratch_types=[
            pltpu.VMEM((2 * _L, _C), jnp.int32),
            pltpu.VMEM((2 * _L, _C, _D), jnp.float32),
            pltpu.VMEM((_C, 2 * _D), jnp.float32),
            pltpu.VMEM((2 * _L, 16), jnp.float32),
            pltpu.SemaphoreType.DMA,
        ],
    )
    def k(idx_hbm, loc_hbm, time_hbm, w_hbm, out_hbm,
          idx_v, rows_v, out_v, w_v, sem):
        wid = lax.axis_index("s") * 2 + lax.axis_index("c")
        pltpu.sync_copy(w_hbm, w_v)
        ws = [w_v[j] for j in range(2 * _L)]

        def chunk_body(g, carry):
            gb = wid * _NCHW + g
            base = gb * _C
            pltpu.sync_copy(idx_hbm.at[gb], idx_v)
            copies = []
            for j in range(_L):
                copies.append(pltpu.async_copy(
                    loc_hbm.at[idx_v.at[j]], rows_v.at[j], sem))
            for j in range(_L):
                copies.append(pltpu.async_copy(
                    time_hbm.at[idx_v.at[_L + j]], rows_v.at[_L + j], sem))
            for cp in copies:
                cp.wait()

            def tok(c, carry2):
                for p in range(2):
                    sl = p * 16
                    a = ws[0] * rows_v[0, c, pl.ds(sl, 16)]
                    for j in range(1, _L):
                        a = a + ws[j] * rows_v[j, c, pl.ds(sl, 16)]
                    out_v[c, pl.ds(sl, 16)] = a
                    b = ws[_L] * rows_v[_L, c, pl.ds(sl, 16)]
                    for j in range(1, _L):
                        b = b + ws[_L + j] * rows_v[_L + j, c, pl.ds(sl, 16)]
                    out_v[c, pl.ds(_D + sl, 16)] = b
                return carry2

            lax.fori_loop(0, _C, tok, 0)
            pltpu.sync_copy(out_v, out_hbm.at[pl.ds(base, _C), :])
            return carry

        lax.fori_loop(0, _NCHW, chunk_body, 0)

    return k


_k = _make_kernel()


def kernel(x, t, loc_tables, time_tables, x_weights, t_weights):
    xf = x.reshape(_N, _L).astype(jnp.int32)
    tf = t.reshape(_N, _L).astype(jnp.int32)
    xl = xf.T + (jnp.arange(_L, dtype=jnp.int32) * _VL)[:, None]
    tl = tf.T + (jnp.arange(_L, dtype=jnp.int32) * _VT)[:, None]
    nch = _N // _C
    xi = xl.reshape(_L, nch, _C).transpose(1, 0, 2)
    ti = tl.reshape(_L, nch, _C).transpose(1, 0, 2)
    idx_all = jnp.concatenate([xi, ti], axis=1)
    loc_flat = loc_tables.reshape(_L * _VL, _D)
    time_flat = time_tables.reshape(_L * _VT, _D)
    w_all = jnp.broadcast_to(
        jnp.concatenate([x_weights, t_weights])[:, None], (2 * _L, 16))
    out = _k(idx_all, loc_flat, time_flat, w_all)
    return out.reshape(_B, _S, 2 * _D)


# speedup vs baseline: 11.8464x; 1.3989x over previous
import functools

import jax
import jax.numpy as jnp
from jax import lax
from jax.experimental import pallas as pl
from jax.experimental.pallas import tpu as pltpu
from jax.experimental.pallas import tpu_sc as plsc

_B, _S = 1024, 200
_L = 4
_VL, _VT = 100000, 512
_D = 32
_N = _B * _S
_NW = 32
_C = 128
_TW = _N // _NW
_NCHW = _TW // _C


def _make_kernel():
    mesh = plsc.VectorSubcoreMesh(core_axis_name="c", subcore_axis_name="s")

    @functools.partial(
        pl.kernel,
        mesh=mesh,
        out_type=jax.ShapeDtypeStruct((_N, 2 * _D), jnp.float32),
        compiler_params=pltpu.CompilerParams(use_tc_tiling_on_sc=False),
        scratch_types=[
            pltpu.VMEM((2, 2 * _L, _C), jnp.int32),
            pltpu.VMEM((2, 2 * _L, _C, _D), jnp.float32),
            pltpu.VMEM((2, _C, 2 * _D), jnp.float32),
            pltpu.VMEM((2 * _L, 16), jnp.float32),
            pltpu.SemaphoreType.DMA,
            pltpu.SemaphoreType.DMA,
            pltpu.SemaphoreType.DMA,
            pltpu.SemaphoreType.DMA,
            pltpu.SemaphoreType.DMA,
            pltpu.SemaphoreType.DMA,
        ],
    )
    def k(idx_hbm, loc_hbm, time_hbm, w_hbm, out_hbm,
          idx_v, rows_v, out_v, w_v,
          sem_i0, sem_i1, sem_g0, sem_g1, sem_o0, sem_o1):
        wid = lax.axis_index("s") * 2 + lax.axis_index("c")
        g0 = wid * _NCHW
        sem_i = [sem_i0, sem_i1]
        sem_g = [sem_g0, sem_g1]
        sem_o = [sem_o0, sem_o1]

        pltpu.sync_copy(w_hbm, w_v)
        ws = [w_v[j] for j in range(2 * _L)]

        def issue_gathers(par):
            cps = []
            for j in range(_L):
                cps.append(pltpu.async_copy(
                    loc_hbm.at[idx_v.at[par, j]], rows_v.at[par, j],
                    sem_g[par]))
            for j in range(_L):
                cps.append(pltpu.async_copy(
                    time_hbm.at[idx_v.at[par, _L + j]],
                    rows_v.at[par, _L + j], sem_g[par]))
            return cps

        def wait_gathers(par):
            for j in range(2 * _L):
                pltpu.make_async_copy(
                    loc_hbm.at[pl.ds(0, _C)], rows_v.at[par, j],
                    sem_g[par]).wait()

        def compute(par):
            @plsc.parallel_loop(0, _C, unroll=4)
            def tok(c):
                for p in range(2):
                    sl = p * 16
                    a = ws[0] * rows_v[par, 0, c, pl.ds(sl, 16)]
                    for j in range(1, _L):
                        a = a + ws[j] * rows_v[par, j, c, pl.ds(sl, 16)]
                    out_v[par, c, pl.ds(sl, 16)] = a
                    b = ws[_L] * rows_v[par, _L, c, pl.ds(sl, 16)]
                    for j in range(1, _L):
                        b = b + ws[_L + j] * rows_v[par, _L + j, c,
                                                    pl.ds(sl, 16)]
                    out_v[par, c, pl.ds(_D + sl, 16)] = b

        def out_slice(g):
            return out_hbm.at[pl.ds(g * _C, _C), :]

        pltpu.sync_copy(idx_hbm.at[g0], idx_v.at[0])
        pltpu.async_copy(idx_hbm.at[g0 + 1], idx_v.at[1], sem_i[1])
        issue_gathers(0)

        def super_body(i, carry):
            for par in range(2):
                g = g0 + 2 * i + par
                wait_gathers(par)

                @pl.when(i < _NCHW // 2 - 1)
                def _prefetch_idx():
                    pltpu.async_copy(idx_hbm.at[g + 2], idx_v.at[par],
                                     sem_i[par])

                nxt = 1 - par

                def _launch_next():
                    pltpu.make_async_copy(
                        idx_hbm.at[g + 1], idx_v.at[nxt], sem_i[nxt]).wait()
                    issue_gathers(nxt)

                if par == 0:
                    _launch_next()
                else:
                    pl.when(i < _NCHW // 2 - 1)(_launch_next)

                @pl.when(i > 0)
                def _drain_out():
                    pltpu.make_async_copy(
                        out_v.at[par], out_slice(g - 2), sem_o[par]).wait()

                compute(par)
                pltpu.async_copy(out_v.at[par], out_slice(g), sem_o[par])
            return carry

        lax.fori_loop(0, _NCHW // 2, super_body, 0)

        last = g0 + _NCHW - 2
        pltpu.make_async_copy(out_v.at[0], out_slice(last), sem_o[0]).wait()
        pltpu.make_async_copy(out_v.at[1], out_slice(last + 1),
                              sem_o[1]).wait()

    return k


_k = _make_kernel()


def kernel(x, t, loc_tables, time_tables, x_weights, t_weights):
    xf = x.reshape(_N, _L).astype(jnp.int32)
    tf = t.reshape(_N, _L).astype(jnp.int32)
    xl = xf.T + (jnp.arange(_L, dtype=jnp.int32) * _VL)[:, None]
    tl = tf.T + (jnp.arange(_L, dtype=jnp.int32) * _VT)[:, None]
    nch = _N // _C
    xi = xl.reshape(_L, nch, _C).transpose(1, 0, 2)
    ti = tl.reshape(_L, nch, _C).transpose(1, 0, 2)
    idx_all = jnp.concatenate([xi, ti], axis=1)
    loc_flat = loc_tables.reshape(_L * _VL, _D)
    time_flat = time_tables.reshape(_L * _VT, _D)
    w_all = jnp.broadcast_to(
        jnp.concatenate([x_weights, t_weights])[:, None], (2 * _L, 16))
    out = _k(idx_all, loc_flat, time_flat, w_all)
    return out.reshape(_B, _S, 2 * _D)
